# baseline (speedup 1.0000x reference)
import functools
import math

import jax
import jax.numpy as jnp
from jax import lax
from jax.experimental import pallas as pl
from jax.experimental.pallas import tpu as pltpu

EPS = 1e-5


def _corr_kernel(blk, k, m, ow_pad, zwT_ref, x_ref, o_ref, xc_ref):
    xc_ref[...] = x_ref[...]
    w1 = m * m - (k - 1)
    for b in range(blk):
        sx = jnp.sqrt(x_ref[b].astype(jnp.float32))
        zwT = zwT_ref[b].astype(jnp.float32)
        d = lax.dot_general(zwT, sx, (((1,), (0,)), ((), ())),
                            preferred_element_type=jnp.float32)
        acc = d[0:k, 0:w1]
        for j in range(1, k):
            acc = acc + d[j * k:(j + 1) * k, j:j + w1]
        acc = jnp.concatenate(
            [acc, jnp.zeros((k, k - 1), jnp.float32)], axis=1)
        out = acc[0:1, 0:ow_pad]
        for i in range(1, k):
            out = out + acc[i:i + 1, i * m:i * m + ow_pad]
        o_ref[b] = out


def _bn_kernel(total, ow, m, gb_ref, c_ref, o_ref):
    x = c_ref[...]
    lane = lax.broadcasted_iota(jnp.int32, x.shape, 1)
    mask = (lane % m) < ow
    inv_n = 1.0 / float(total)
    mean = jnp.sum(jnp.where(mask, x, 0.0)) * inv_n
    dev = jnp.where(mask, x - mean, 0.0)
    var = jnp.sum(dev * dev) * inv_n
    inv_std = lax.rsqrt(var + EPS)
    scale = gb_ref[0] * inv_std
    shift = gb_ref[1] - mean * scale
    o_ref[...] = (x * scale + shift).astype(o_ref.dtype)


def kernel(z_feat, x_feat, bc_weights, bn_gamma, bn_beta):
    N, C, k, _ = z_feat.shape
    m = x_feat.shape[2]
    oh = ow = m - k + 1
    kk = k * k
    ow_pad = oh * m
    inv_k2 = 1.0 / float(k * k)
    blk = max(b for b in (1, 2, 4, 8) if N % b == 0)

    zw = (bc_weights.reshape(1, C, 1, 1).astype(jnp.float32) * inv_k2) * jnp.sqrt(
        z_feat.astype(jnp.float32))
    zwT = zw.transpose(0, 3, 2, 1).reshape(N, kk, C)
    x2 = x_feat.reshape(N, C, m * m)

    coeff, xcopy = pl.pallas_call(
        functools.partial(_corr_kernel, blk, k, m, ow_pad),
        out_shape=(jax.ShapeDtypeStruct((N, 1, ow_pad), jnp.float32),
                   jax.ShapeDtypeStruct((N, C, m * m), x_feat.dtype)),
        grid=(N // blk,),
        in_specs=[pl.BlockSpec((blk, kk, C), lambda n: (n, 0, 0)),
                  pl.BlockSpec((blk, C, m * m), lambda n: (n, 0, 0))],
        out_specs=(pl.BlockSpec((blk, 1, ow_pad), lambda n: (n, 0, 0)),
                   pl.BlockSpec((blk, C, m * m), lambda n: (n, 0, 0))),
        compiler_params=pltpu.CompilerParams(
            dimension_semantics=("parallel",),
            vmem_limit_bytes=56 * 1024 * 1024),
    )(zwT, x2)

    gb = jnp.stack([bn_gamma.reshape(()).astype(jnp.float32),
                    bn_beta.reshape(()).astype(jnp.float32)])
    total = N * oh * ow
    out2 = pl.pallas_call(
        functools.partial(_bn_kernel, total, ow, m),
        out_shape=jax.ShapeDtypeStruct((N, ow_pad), x_feat.dtype),
        in_specs=[pl.BlockSpec(memory_space=pltpu.SMEM),
                  pl.BlockSpec(memory_space=pltpu.VMEM)],
        out_specs=pl.BlockSpec(memory_space=pltpu.VMEM),
        compiler_params=pltpu.CompilerParams(
            vmem_limit_bytes=32 * 1024 * 1024),
    )(gb, coeff.reshape(N, ow_pad))

    out = out2.reshape(N, oh, m)[:, :, :ow].reshape(N, 1, oh, ow)
    return out, xcopy.reshape(x_feat.shape), z_feat


# speedup vs baseline: 1.5859x; 1.5859x over previous
import functools
import math

import jax
import jax.numpy as jnp
from jax import lax
from jax.experimental import pallas as pl
from jax.experimental.pallas import tpu as pltpu

EPS = 1e-5


def _corr_kernel(blk, k, m, ow_pad, zwT_ref, x_ref, o_ref):
    w1 = m * m - (k - 1)
    oh = ow = m - k + 1
    w2 = (oh - 1) * m + ow
    for b in range(blk):
        x = x_ref[b].astype(jnp.float32).reshape(x_ref.shape[1], m * m)
        sx = x * lax.rsqrt(jnp.maximum(x, 1e-30))
        zwT = zwT_ref[b].astype(jnp.float32)
        d = lax.dot_general(zwT, sx, (((1,), (0,)), ((), ())),
                            preferred_element_type=jnp.float32)
        acc = d[0:k, 0:w1]
        for j in range(1, k):
            acc = acc + d[j * k:(j + 1) * k, j:j + w1]
        out = acc[0:1, 0:w2]
        for i in range(1, k):
            out = out + acc[i:i + 1, i * m:i * m + w2]
        o_ref[b, :, 0:w2] = out
        o_ref[b, :, w2:ow_pad] = jnp.zeros((1, ow_pad - w2), jnp.float32)


def _bn_kernel(total, ow, m, gb_ref, c_ref, o_ref):
    x = c_ref[...]
    lane = lax.broadcasted_iota(jnp.int32, x.shape, 1)
    mask = (lane % m) < ow
    inv_n = 1.0 / float(total)
    mean = jnp.sum(jnp.where(mask, x, 0.0)) * inv_n
    dev = jnp.where(mask, x - mean, 0.0)
    var = jnp.sum(dev * dev) * inv_n
    inv_std = lax.rsqrt(var + EPS)
    scale = gb_ref[0] * inv_std
    shift = gb_ref[1] - mean * scale
    o_ref[...] = (x * scale + shift).astype(o_ref.dtype)


def kernel(z_feat, x_feat, bc_weights, bn_gamma, bn_beta):
    N, C, k, _ = z_feat.shape
    m = x_feat.shape[2]
    oh = ow = m - k + 1
    kk = k * k
    ow_pad = oh * m
    inv_k2 = 1.0 / float(k * k)
    blk = max(b for b in (1, 2, 4, 8) if N % b == 0)

    zw = (bc_weights.reshape(1, C, 1, 1).astype(jnp.float32) * inv_k2) * jnp.sqrt(
        z_feat.astype(jnp.float32))
    zwT = zw.transpose(0, 3, 2, 1).reshape(N, kk, C)

    coeff = pl.pallas_call(
        functools.partial(_corr_kernel, blk, k, m, ow_pad),
        out_shape=jax.ShapeDtypeStruct((N, 1, ow_pad), jnp.float32),
        grid=(N // blk,),
        in_specs=[pl.BlockSpec((blk, kk, C), lambda n: (n, 0, 0)),
                  pl.BlockSpec((blk, C, m, m), lambda n: (n, 0, 0, 0))],
        out_specs=pl.BlockSpec((blk, 1, ow_pad), lambda n: (n, 0, 0)),
        compiler_params=pltpu.CompilerParams(
            dimension_semantics=("parallel",),
            vmem_limit_bytes=48 * 1024 * 1024),
    )(zwT, x_feat)

    gb = jnp.stack([bn_gamma.reshape(()).astype(jnp.float32),
                    bn_beta.reshape(()).astype(jnp.float32)])
    total = N * oh * ow
    out2 = pl.pallas_call(
        functools.partial(_bn_kernel, total, ow, m),
        out_shape=jax.ShapeDtypeStruct((N, ow_pad), x_feat.dtype),
        in_specs=[pl.BlockSpec(memory_space=pltpu.SMEM),
                  pl.BlockSpec(memory_space=pltpu.VMEM)],
        out_specs=pl.BlockSpec(memory_space=pltpu.VMEM),
        compiler_params=pltpu.CompilerParams(
            vmem_limit_bytes=32 * 1024 * 1024),
    )(gb, coeff.reshape(N, ow_pad))

    out = out2.reshape(N, oh, m)[:, :, :ow].reshape(N, 1, oh, ow)
    return out, x_feat, z_feat
